# baseline (speedup 1.0000x reference)
import functools

import jax
import jax.numpy as jnp
from jax import lax
from jax.experimental import pallas as pl
from jax.experimental.pallas import tpu as pltpu
from jax.experimental.pallas import tpu_sc as plsc

_SL, _B, _V, _D = 200, 4096, 1000000, 64
_NW = 32
_N = _SL * _B
_PER_W = _N // _NW
_C = 128
_NCH = _PER_W // _C
_L = 16
_NB = 4
_V2 = _V // 2
_BT = _B // _C

_mesh = plsc.VectorSubcoreMesh(core_axis_name="c", subcore_axis_name="s")


@functools.partial(
    pl.kernel,
    mesh=_mesh,
    compiler_params=pltpu.CompilerParams(needs_layout_passes=False),
    out_type=jax.ShapeDtypeStruct((_SL * 8, _BT, 8, 128), jnp.float32),
    scratch_types=[
        pltpu.VMEM((_NB, _C), jnp.int32),
        pltpu.VMEM((_NB, _C), jnp.int32),
        pltpu.VMEM((_NB, _C), jnp.int32),
        pltpu.VMEM((_SL * _D,), jnp.float32),
        pltpu.VMEM((_NB, _C, 128), jnp.float32),
        pltpu.VMEM((_NB, 8, 1, 8, 128), jnp.float32),
        pltpu.SemaphoreType.DMA((_NB,)),
        pltpu.SemaphoreType.DMA((_NB,)),
        pltpu.SemaphoreType.DMA((_NB,)),
    ],
)
def _embed(x_hbm, tok_hbm, pos_hbm, out_hbm,
           xfc, idx2, par2, pos_v, wide, outt, xsem, gsem, osem):
    wid = lax.axis_index("s") * 2 + lax.axis_index("c")
    base = wid * _PER_W
    pltpu.sync_copy(pos_hbm, pos_v)

    iota = lax.iota(jnp.int32, _L)
    zeros = iota * 0
    rowv = [g * _L + iota for g in range(_C // _L)]

    def xdma(g, b):
        return pltpu.make_async_copy(
            x_hbm.at[pl.ds(base + g * _C, _C)], xfc.at[b], xsem.at[b]
        )

    def gather(g, b):
        return pltpu.make_async_copy(
            tok_hbm.at[idx2.at[b]], wide.at[b], gsem.at[b]
        )

    def store(g, b):
        start = base + g * _C
        sl = start // _B
        bt = (start % _B) // _C
        return pltpu.make_async_copy(
            outt.at[b], out_hbm.at[pl.ds(sl * 8, 8), pl.ds(bt, 1)], osem.at[b]
        )

    def fill(b):
        for t in range(_C // _L):
            xv = xfc[b, pl.ds(t * _L, _L)]
            idx2[b, pl.ds(t * _L, _L)] = lax.shift_right_logical(xv, 1)
            par2[b, pl.ds(t * _L, _L)] = lax.shift_left(xv & 1, 6)

    for b in range(_NB):
        xdma(b, b).start()
    for b in range(_NB):
        xdma(b, b).wait()
        fill(b)
        gather(b, b).start()
        xdma(b + _NB, b).start()

    def group_body(i, carry):
        g0 = i * _NB
        for b in range(_NB):
            g = g0 + b
            gather(g, b).wait()

            @pl.when(g >= _NB)
            def _():
                store(g - _NB, b).wait()

            start = base + g * _C
            sl = start // _B
            pos_base = sl * _D
            colb = [par2[b, pl.ds(gg * _L, _L)] for gg in range(_C // _L)]

            @plsc.parallel_loop(0, _D, step=1, unroll=4)
            def _dcol(d, b=b, colb=colb, pos_base=pos_base):
                dt = lax.shift_right_logical(d, 3)
                s = d & 7
                pv = plsc.load_gather(pos_v, [zeros + (pos_base + d)])
                for gg in range(_C // _L):
                    v = plsc.load_gather(wide.at[b], [rowv[gg], colb[gg] + d])
                    outt[b, dt, 0, s, pl.ds(gg * _L, _L)] = v + pv

            store(g, b).start()

            gn = g + _NB

            @pl.when(gn < _NCH)
            def _():
                xdma(gn, b).wait()
                fill(b)
                gather(gn, b).start()

                @pl.when(gn + _NB < _NCH)
                def _():
                    xdma(gn + _NB, b).start()

        return carry

    lax.fori_loop(0, _NCH // _NB, group_body, 0)

    for b in range(_NB):
        store(_NCH - _NB + b, b).wait()


def kernel(x, token_table, pos_table):
    out4 = _embed(
        x.reshape(_N),
        token_table.reshape(_V2, 128),
        pos_table.reshape(_SL * _D),
    )
    o = out4.reshape(_SL, 8, _BT, 8, 128)
    return o.transpose(0, 2, 4, 1, 3).reshape(_SL, _B, _D)


# speedup vs baseline: 1.0695x; 1.0695x over previous
import functools

import jax
import jax.numpy as jnp
from jax import lax
from jax.experimental import pallas as pl
from jax.experimental.pallas import tpu as pltpu
from jax.experimental.pallas import tpu_sc as plsc

_SL, _B, _V, _D = 200, 4096, 1000000, 64
_NW = 32
_N = _SL * _B
_PER_W = _N // _NW
_C = 128
_NCH = _PER_W // _C
_L = 16
_NB = 8
_RU = 4
_LAG = 3

_mesh = plsc.VectorSubcoreMesh(core_axis_name="c", subcore_axis_name="s")


@functools.partial(
    pl.kernel,
    mesh=_mesh,
    compiler_params=pltpu.CompilerParams(use_tc_tiling_on_sc=False),
    out_type=jax.ShapeDtypeStruct((_N, _D), jnp.float32),
    scratch_types=[
        pltpu.VMEM((_PER_W,), jnp.int32),
        pltpu.VMEM((_SL, _D), jnp.float32),
        pltpu.VMEM((_NB, _C, _D), jnp.float32),
        pltpu.SemaphoreType.DMA((_NB,)),
        pltpu.SemaphoreType.DMA((_NB,)),
    ],
)
def _embed(x_hbm, tok_hbm, pos_hbm, out_hbm, idx_v, pos_v, bufs, gsem, osem):
    wid = lax.axis_index("s") * 2 + lax.axis_index("c")
    base = wid * _PER_W
    pltpu.sync_copy(x_hbm.at[pl.ds(base, _PER_W)], idx_v)
    pltpu.sync_copy(pos_hbm, pos_v)

    def gather(g, b):
        return pltpu.make_async_copy(
            tok_hbm.at[idx_v.at[pl.ds(g * _C, _C)]], bufs.at[b], gsem.at[b]
        )

    def store(g, b):
        return pltpu.make_async_copy(
            bufs.at[b], out_hbm.at[pl.ds(base + g * _C, _C)], osem.at[b]
        )

    for b in range(_NB):
        gather(b, b).start()

    def group_body(i, carry):
        g0 = i * _NB
        for b in range(_NB):
            g = g0 + b
            gather(g, b).wait()
            sl = (base + g * _C) // _B
            pvecs = [pos_v[sl, pl.ds(j * _L, _L)] for j in range(_D // _L)]

            @plsc.parallel_loop(0, _C, step=1, unroll=_RU)
            def _row(r, b=b, pvecs=pvecs):
                for j in range(_D // _L):
                    plsc.addupdate(bufs.at[b, r, pl.ds(j * _L, _L)], pvecs[j])

            store(g, b).start()

            bp = (b - _LAG) % _NB
            gp = g - _LAG

            @pl.when((gp >= 0) & (gp + _NB < _NCH))
            def _():
                store(gp, bp).wait()
                gather(gp + _NB, bp).start()

        return carry

    lax.fori_loop(0, _NCH // _NB, group_body, 0)

    for b in range(_NB):
        store(_NCH - _NB + b, b).wait()


def kernel(x, token_table, pos_table):
    out = _embed(x.reshape(_N), token_table, pos_table)
    return out.reshape(_SL, _B, _D)
